# baseline (speedup 1.0000x reference)
import jax
import jax.numpy as jnp
from jax.experimental import pallas as pl
from jax.experimental.pallas import tpu as pltpu

_BLOCK_ROWS = 2048


def _copy_body(w_ref, o_ref):
    o_ref[...] = w_ref[...]


def kernel(indices, weight):
    seq_len = indices.shape[-2]
    cols = weight.shape[1]
    br = min(_BLOCK_ROWS, seq_len)
    while seq_len % br:
        br //= 2
    grid = seq_len // br
    return pl.pallas_call(
        _copy_body,
        grid=(grid,),
        out_shape=jax.ShapeDtypeStruct((seq_len, cols), weight.dtype),
        in_specs=[pl.BlockSpec((br, cols), lambda i: (i, 0))],
        out_specs=pl.BlockSpec((br, cols), lambda i: (i, 0)),
    )(weight)


# speedup vs baseline: 1.2238x; 1.2238x over previous
import jax
import jax.numpy as jnp
from jax.experimental import pallas as pl
from jax.experimental.pallas import tpu as pltpu

_NCHUNK = 4


def _copy_body(seq_len, cols, nchunk):
    rows = seq_len // nchunk

    def body(w_hbm, o_hbm, vmem, rsem, wsem):
        reads = []
        for i in range(nchunk):
            sl = pl.ds(i * rows, rows)
            reads.append(pltpu.make_async_copy(w_hbm.at[sl, :], vmem.at[i], rsem.at[i]))
        for r in reads:
            r.start()
        writes = []
        for i in range(nchunk):
            sl = pl.ds(i * rows, rows)
            reads[i].wait()
            w = pltpu.make_async_copy(vmem.at[i], o_hbm.at[sl, :], wsem.at[i])
            w.start()
            writes.append(w)
        for w in writes:
            w.wait()

    return body


def kernel(indices, weight):
    seq_len = indices.shape[-2]
    cols = weight.shape[1]
    nchunk = _NCHUNK
    while seq_len % nchunk:
        nchunk //= 2
    rows = seq_len // nchunk
    return pl.pallas_call(
        _copy_body(seq_len, cols, nchunk),
        out_shape=jax.ShapeDtypeStruct((seq_len, cols), weight.dtype),
        in_specs=[pl.BlockSpec(memory_space=pl.ANY)],
        out_specs=pl.BlockSpec(memory_space=pl.ANY),
        scratch_shapes=[
            pltpu.VMEM((nchunk, rows, cols), weight.dtype),
            pltpu.SemaphoreType.DMA((nchunk,)),
            pltpu.SemaphoreType.DMA((nchunk,)),
        ],
    )(weight)
